# baseline (speedup 1.0000x reference)
import functools

import jax
import jax.numpy as jnp
from jax import lax
from jax.experimental import pallas as pl
from jax.experimental.pallas import tpu as pltpu
from jax.experimental.pallas import tpu_sc as plsc

D_MODEL = 1024
MAX_REL = 128


def _make_pos_embed_sc(seq_len: int):
    B = 2 * seq_len - 1
    B_pad = 2 * seq_len
    info = plsc.get_sparse_core_info()
    NC, NS, L = info.num_cores, info.num_subcores, info.num_lanes
    NW = NC * NS
    b_per_w = B_pad // NW
    CHUNK = 64
    n_chunks = b_per_w // CHUNK

    mesh = plsc.VectorSubcoreMesh(core_axis_name="c", subcore_axis_name="s")

    @functools.partial(
        pl.kernel,
        mesh=mesh,
        out_type=jax.ShapeDtypeStruct((B, D_MODEL), jnp.float32),
        scratch_types=[
            pltpu.VMEM((CHUNK,), jnp.int32),
            pltpu.VMEM((CHUNK,), jnp.int32),
            pltpu.VMEM((CHUNK, D_MODEL), jnp.float32),
            pltpu.SemaphoreType.DMA,
        ],
    )
    def pos_embed_kernel(pe_hbm, out_hbm, idx_v, tidx_v, rows_v, sem):
        wid = lax.axis_index("s") * NC + lax.axis_index("c")
        base_w = wid * b_per_w
        for c in range(n_chunks):
            base = base_w + c * CHUNK
            for j in range(CHUNK // L):
                r = lax.iota(jnp.int32, L) + (base + j * L)
                idx = jnp.clip(r - (seq_len - 1), -MAX_REL, MAX_REL) + MAX_REL
                idx_v[pl.ds(j * L, L)] = idx
            pltpu.async_copy(pe_hbm.at[idx_v], rows_v, sem).wait()
            is_tail = base == B_pad - CHUNK

            @pl.when(jnp.logical_not(is_tail))
            def _():
                pltpu.sync_copy(rows_v, out_hbm.at[pl.ds(base, CHUNK)])

            @pl.when(is_tail)
            def _():
                for j in range(CHUNK // L):
                    rr = lax.iota(jnp.int32, L) + (base + j * L)
                    tidx_v[pl.ds(j * L, L)] = jnp.minimum(rr, B - 1)
                pltpu.async_copy(rows_v, out_hbm.at[tidx_v], sem).wait()

    return pos_embed_kernel


def kernel(x, pe):
    seq_len = x.shape[1]
    pos_embed = _make_pos_embed_sc(seq_len)(pe)
    return (x, pos_embed)


# speedup vs baseline: 3.0605x; 3.0605x over previous
import functools

import jax
import jax.numpy as jnp
from jax import lax
from jax.experimental import pallas as pl
from jax.experimental.pallas import tpu as pltpu
from jax.experimental.pallas import tpu_sc as plsc

D_MODEL = 1024
MAX_REL = 128


def _make_pos_embed_sc(seq_len: int):
    B = 2 * seq_len - 1
    B_pad = 2 * seq_len
    info = plsc.get_sparse_core_info()
    NC, NS, L = info.num_cores, info.num_subcores, info.num_lanes
    NW = NC * NS
    b_per_w = B_pad // NW
    CHUNK = 64
    n_chunks = b_per_w // CHUNK
    dist = seq_len - 1

    mesh = plsc.VectorSubcoreMesh(core_axis_name="c", subcore_axis_name="s")

    @functools.partial(
        pl.kernel,
        mesh=mesh,
        out_type=jax.ShapeDtypeStruct((B, D_MODEL), jnp.float32),
        scratch_types=[
            pltpu.VMEM((L,), jnp.int32),
            pltpu.VMEM((CHUNK,), jnp.int32),
            pltpu.VMEM((L,), jnp.int32),
            pltpu.VMEM((CHUNK, D_MODEL), jnp.float32),
            pltpu.SemaphoreType.DMA,
            pltpu.SemaphoreType.DMA,
        ],
    )
    def pos_embed_kernel(pe_hbm, out_hbm, cidx_v, gidx_v, tidx_v, rows_v,
                         gsem, osem):
        wid = lax.axis_index("s") * NC + lax.axis_index("c")
        base_w = wid * b_per_w
        i_first = jnp.clip(base_w - dist, -MAX_REL, MAX_REL) + MAX_REL
        i_last = jnp.clip(base_w + b_per_w - 1 - dist, -MAX_REL, MAX_REL) + MAX_REL
        span_const = i_first == i_last

        @pl.when(span_const)
        def _():
            cidx_v[...] = jnp.zeros((L,), jnp.int32) + i_first
            pltpu.async_copy(pe_hbm.at[cidx_v], rows_v.at[pl.ds(0, L)],
                             gsem).wait()
            is_last_w = wid == NW - 1
            n16 = b_per_w // L

            @pl.when(jnp.logical_not(is_last_w))
            def _():
                copies = [
                    pltpu.async_copy(
                        rows_v.at[pl.ds(0, L)],
                        out_hbm.at[pl.ds(base_w + c * L, L)], osem)
                    for c in range(n16)
                ]
                for cp in copies:
                    cp.wait()

            @pl.when(is_last_w)
            def _():
                copies = [
                    pltpu.async_copy(
                        rows_v.at[pl.ds(0, L)],
                        out_hbm.at[pl.ds(base_w + c * L, L)], osem)
                    for c in range(n16 - 1)
                ]
                tail = base_w + (n16 - 1) * L
                copies.append(pltpu.async_copy(
                    rows_v.at[pl.ds(0, 8)], out_hbm.at[pl.ds(tail, 8)],
                    osem))
                rr = lax.iota(jnp.int32, L) + (B_pad - L)
                tidx_v[...] = jnp.minimum(rr, B - 1)
                copies.append(pltpu.async_copy(
                    rows_v.at[pl.ds(0, L)], out_hbm.at[tidx_v], osem))
                for cp in copies:
                    cp.wait()

        @pl.when(jnp.logical_not(span_const))
        def _():
            for c in range(n_chunks):
                base = base_w + c * CHUNK
                for j in range(CHUNK // L):
                    r = lax.iota(jnp.int32, L) + (base + j * L)
                    gidx_v[pl.ds(j * L, L)] = (
                        jnp.clip(r - dist, -MAX_REL, MAX_REL) + MAX_REL)
                pltpu.async_copy(pe_hbm.at[gidx_v], rows_v, gsem).wait()
                pltpu.sync_copy(rows_v, out_hbm.at[pl.ds(base, CHUNK)])

    return pos_embed_kernel


def kernel(x, pe):
    seq_len = x.shape[1]
    pos_embed = _make_pos_embed_sc(seq_len)(pe)
    return (x, pos_embed)


# speedup vs baseline: 5.4400x; 1.7775x over previous
import functools

import jax
import jax.numpy as jnp
from jax import lax
from jax.experimental import pallas as pl
from jax.experimental.pallas import tpu as pltpu
from jax.experimental.pallas import tpu_sc as plsc

D_MODEL = 1024
MAX_REL = 128


def _make_pos_embed_sc(seq_len: int):
    B = 2 * seq_len - 1
    B_pad = 2 * seq_len
    info = plsc.get_sparse_core_info()
    NC, NS, L = info.num_cores, info.num_subcores, info.num_lanes
    NW = NC * NS
    b_per_w = B_pad // NW
    CHUNK = 64
    n_chunks = b_per_w // CHUNK
    dist = seq_len - 1

    lo, hi = dist - MAX_REL, dist + MAX_REL
    for w in range(NW):
        has0 = any(w * b_per_w + c * CHUNK + CHUNK - 1 < lo
                   for c in range(n_chunks))
        has1 = any(w * b_per_w + c * CHUNK > hi for c in range(n_chunks))
        assert not (has0 and has1), "worker spans both clip regions"

    mesh = plsc.VectorSubcoreMesh(core_axis_name="c", subcore_axis_name="s")

    @functools.partial(
        pl.kernel,
        mesh=mesh,
        out_type=jax.ShapeDtypeStruct((B, D_MODEL), jnp.float32),
        scratch_types=[
            pltpu.VMEM((CHUNK,), jnp.int32),
            pltpu.VMEM((L,), jnp.int32),
            pltpu.VMEM((32, D_MODEL), jnp.float32),
            pltpu.VMEM((CHUNK, D_MODEL), jnp.float32),
            pltpu.SemaphoreType.DMA,
            pltpu.SemaphoreType.DMA,
        ],
    )
    def pos_embed_kernel(amp_hbm, pe_hbm, out_hbm, gidx_v, tidx_v, rep_v,
                         big_v, gsem, osem):
        wid = lax.axis_index("s") * NC + lax.axis_index("c")
        base_w = wid * b_per_w

        def table_idx(r):
            return jnp.clip(r - dist, -MAX_REL, MAX_REL) + MAX_REL

        i_first = table_idx(base_w)
        chunk0_const = i_first == table_idx(base_w + CHUNK - 1)
        amp_row = jnp.where(chunk0_const, i_first,
                            table_idx(base_w + b_per_w - 1))
        amp_off = jnp.where(amp_row == 0, 0, 32)

        pltpu.sync_copy(amp_hbm.at[pl.ds(amp_off, 32)], rep_v)

        for c in range(n_chunks):
            cbase = base_w + c * CHUNK
            c_const = table_idx(cbase) == table_idx(cbase + CHUNK - 1)
            is_tail = cbase == B_pad - CHUNK

            @pl.when(c_const & jnp.logical_not(is_tail))
            def _():
                cps = [
                    pltpu.async_copy(
                        rep_v, out_hbm.at[pl.ds(cbase + 32 * k, 32)], osem)
                    for k in range(CHUNK // 32)
                ]
                for cp in cps:
                    cp.wait()

            @pl.when(c_const & is_tail)
            def _():
                cps = [pltpu.async_copy(
                    rep_v, out_hbm.at[pl.ds(cbase, 32)], osem)]
                cps.append(pltpu.async_copy(
                    rep_v.at[pl.ds(0, 16)],
                    out_hbm.at[pl.ds(cbase + 32, 16)], osem))
                cps.append(pltpu.async_copy(
                    rep_v.at[pl.ds(0, 8)],
                    out_hbm.at[pl.ds(cbase + 48, 8)], osem))
                rr = lax.iota(jnp.int32, L) + (B_pad - L)
                tidx_v[...] = jnp.minimum(rr, B - 1)
                cps.append(pltpu.async_copy(
                    rep_v.at[pl.ds(0, L)], out_hbm.at[tidx_v], osem))
                for cp in cps:
                    cp.wait()

            @pl.when(jnp.logical_not(c_const))
            def _():
                for j in range(CHUNK // L):
                    r = lax.iota(jnp.int32, L) + (cbase + j * L)
                    gidx_v[pl.ds(j * L, L)] = table_idx(r)
                pltpu.async_copy(pe_hbm.at[gidx_v], big_v, gsem).wait()
                pltpu.sync_copy(big_v, out_hbm.at[pl.ds(cbase, CHUNK)])

    return pos_embed_kernel


def _x_copy_tc(x):
    b, s, d = x.shape
    blk = 1024
    return pl.pallas_call(
        lambda x_ref, o_ref: o_ref.__setitem__((...,), x_ref[...]),
        grid=(b, s // blk),
        in_specs=[pl.BlockSpec((1, blk, d), lambda i, j: (i, j, 0))],
        out_specs=pl.BlockSpec((1, blk, d), lambda i, j: (i, j, 0)),
        out_shape=jax.ShapeDtypeStruct(x.shape, x.dtype),
    )(x)


def kernel(x, pe):
    seq_len = x.shape[1]
    amp = jnp.concatenate([
        jnp.broadcast_to(pe[0], (32, pe.shape[1])),
        jnp.broadcast_to(pe[2 * MAX_REL], (32, pe.shape[1])),
    ])
    pos_embed = _make_pos_embed_sc(seq_len)(amp, pe)
    x_out = _x_copy_tc(x)
    return (x_out, pos_embed)
